# Initial kernel scaffold; baseline (speedup 1.0000x reference)
import jax
import jax.numpy as jnp
from jax.experimental import pallas as pl


def kernel(x, qk_neurons, v_neurons, know_neurons, neuron_pos, W_pos_qk, b_pos_qk, W_pos_v, b_pos_v, W_pos_know, b_pos_know, W_tau_attn, b_tau_attn, W_tau_know, b_tau_know, deterministic):
    raise NotImplementedError("write your pallas kernel here")


# speedup vs baseline: 4.0453x; 4.0453x over previous
import functools

import jax
import jax.numpy as jnp
from jax.experimental import pallas as pl
from jax.experimental.pallas import tpu as pltpu

D_MODEL = 768
POS_DIM = 2
K_CAND = 24
K_TOP = 8
KEEP = 0.9
SEQ = 2048


def _top8_threshold(eg):
    iota = jax.lax.broadcasted_iota(jnp.int32, eg.shape, 1)
    v = eg
    m = None
    for j in range(K_TOP):
        m = jnp.max(v, axis=1, keepdims=True)
        if j < K_TOP - 1:
            idx = jnp.min(jnp.where(v == m, iota, K_CAND), axis=1, keepdims=True)
            v = jnp.where(iota == idx, -jnp.inf, v)
    return m


def _threshold_gate(scores, tau):
    raw = scores - tau
    gate = jnp.where(raw > 0, raw, 1e-08 * jnp.exp(raw))
    eg = jnp.exp(gate) - 1.0
    thr = _top8_threshold(eg)
    eg = jnp.where(eg >= thr, eg, 0.0)
    gsum = jnp.sum(eg, axis=1, keepdims=True) + 1e-08
    gstr = jnp.tanh(jnp.max(eg, axis=1, keepdims=True))
    return eg / gsum * gstr


def _pool_body(x_ref, pos_ref, nt_ref, npt_ref, wt_ref, bt_ref,
               idx_ref, loss_ref, *gate_refs, N, n_tau, t_blk):
    i = pl.program_id(0)
    x = x_ref[...]
    pos = pos_ref[...]
    d0 = pos[:, 0:1] - npt_ref[0:1, :]
    d1 = pos[:, 1:2] - npt_ref[1:2, :]
    dist = d0 * d0 + d1 * d1
    scores = jnp.dot(x / KEEP, nt_ref[...],
                     preferred_element_type=jnp.float32)
    tau = jnp.dot(x, wt_ref[...], preferred_element_type=jnp.float32)
    tau = tau + bt_ref[...]

    iota_n = jax.lax.broadcasted_iota(jnp.int32, (t_blk, N), 1)
    iota_k = jax.lax.broadcasted_iota(jnp.int32, (t_blk, K_CAND), 1)

    def step(j, carry):
        d, acc_i, acc_d, acc_s = carry
        m = jnp.min(d, axis=1, keepdims=True)
        idx = jnp.min(jnp.where(d == m, iota_n, N), axis=1, keepdims=True)
        sel = iota_n == idx
        s = jnp.sum(jnp.where(sel, scores, 0.0), axis=1, keepdims=True)
        d = jnp.where(sel, jnp.inf, d)
        lane = iota_k == j
        acc_i = jnp.where(lane, idx, acc_i)
        acc_d = jnp.where(lane, m, acc_d)
        acc_s = jnp.where(lane, s, acc_s)
        return d, acc_i, acc_d, acc_s

    init = (dist,
            jnp.zeros((t_blk, K_CAND), jnp.int32),
            jnp.zeros((t_blk, K_CAND), jnp.float32),
            jnp.zeros((t_blk, K_CAND), jnp.float32))
    _, cand_i, cand_d, cand_s = jax.lax.fori_loop(0, K_CAND, step, init)

    idx_ref[...] = cand_i
    g0 = None
    for c in range(n_tau):
        g = _threshold_gate(cand_s, tau[:, c:c + 1])
        gate_refs[c][...] = g
        if c == 0:
            g0 = g

    part = jnp.sum(g0 * cand_d, axis=(0, 1), keepdims=True)

    @pl.when(i == 0)
    def _():
        loss_ref[...] = jnp.zeros((1, 1), jnp.float32)

    loss_ref[...] += part


def _run_pool(x2d, pos, neurons, npos, wt, bt, *, n_tau, t_blk):
    N = neurons.shape[0]
    nt = neurons.T
    npt = npos.T
    bt2 = bt.reshape(1, n_tau)
    grid = (SEQ // t_blk,)
    out_shape = ([jax.ShapeDtypeStruct((SEQ, K_CAND), jnp.int32),
                  jax.ShapeDtypeStruct((1, 1), jnp.float32)] +
                 [jax.ShapeDtypeStruct((SEQ, K_CAND), jnp.float32)] * n_tau)
    in_specs = [
        pl.BlockSpec((t_blk, D_MODEL), lambda i: (i, 0)),
        pl.BlockSpec((t_blk, POS_DIM), lambda i: (i, 0)),
        pl.BlockSpec((D_MODEL, N), lambda i: (0, 0)),
        pl.BlockSpec((POS_DIM, N), lambda i: (0, 0)),
        pl.BlockSpec((D_MODEL, n_tau), lambda i: (0, 0)),
        pl.BlockSpec((1, n_tau), lambda i: (0, 0)),
    ]
    out_specs = ([pl.BlockSpec((t_blk, K_CAND), lambda i: (i, 0)),
                  pl.BlockSpec((1, 1), lambda i: (0, 0))] +
                 [pl.BlockSpec((t_blk, K_CAND), lambda i: (i, 0))] * n_tau)
    fn = pl.pallas_call(
        functools.partial(_pool_body, N=N, n_tau=n_tau, t_blk=t_blk),
        grid=grid,
        in_specs=in_specs,
        out_specs=out_specs,
        out_shape=out_shape,
        compiler_params=pltpu.CompilerParams(
            dimension_semantics=("arbitrary",)),
    )
    return fn(x2d, pos, nt, npt, wt, bt2)


def kernel(x, qk_neurons, v_neurons, know_neurons, neuron_pos, W_pos_qk,
           b_pos_qk, W_pos_v, b_pos_v, W_pos_know, b_pos_know, W_tau_attn,
           b_tau_attn, W_tau_know, b_tau_know, deterministic):
    del deterministic
    n_qk = qk_neurons.shape[0]
    n_v = v_neurons.shape[0]
    x2d = x.reshape(SEQ, D_MODEL)

    pos_qk = (x @ W_pos_qk + b_pos_qk).reshape(SEQ, POS_DIM)
    pos_v = (x @ W_pos_v + b_pos_v).reshape(SEQ, POS_DIM)
    pos_know = (x @ W_pos_know + b_pos_know).reshape(SEQ, POS_DIM)

    idx_qk, loss_qk, gate_q, gate_k = _run_pool(
        x2d, pos_qk, qk_neurons, neuron_pos[:n_qk],
        W_tau_attn[:, 0:2], b_tau_attn[0:2], n_tau=2, t_blk=256)
    idx_v, loss_v, gate_v = _run_pool(
        x2d, pos_v, v_neurons, neuron_pos[n_qk:n_qk + n_v],
        W_tau_attn[:, 2:3], b_tau_attn[2:3], n_tau=1, t_blk=256)
    idx_know, loss_know, gate_know = _run_pool(
        x2d, pos_know, know_neurons, neuron_pos[n_qk + n_v:],
        W_tau_know, b_tau_know, n_tau=1, t_blk=256)

    denom = jnp.float32(SEQ * K_CAND)
    pos_loss_attn = (loss_qk[0, 0] + loss_v[0, 0]) / denom
    pos_loss_know = loss_know[0, 0] / denom

    to3 = lambda a: a.reshape(1, SEQ, K_CAND)
    return (to3(gate_q), to3(gate_k), to3(gate_v), to3(idx_qk), to3(idx_v),
            pos_loss_attn, to3(gate_know), to3(idx_know), pos_loss_know)


# speedup vs baseline: 4.6973x; 1.1612x over previous
import functools

import jax
import jax.numpy as jnp
from jax.experimental import pallas as pl
from jax.experimental.pallas import tpu as pltpu

D_MODEL = 768
POS_DIM = 2
K_CAND = 24
K_TOP = 8
KEEP = 0.9
SEQ = 2048


def _top8_threshold(eg):
    v = eg
    cum = jnp.zeros(eg.shape[:1] + (1,), jnp.float32)
    thr = jnp.zeros(eg.shape[:1] + (1,), jnp.float32)
    m1 = None
    for j in range(K_TOP):
        m = jnp.max(v, axis=1, keepdims=True)
        if j == 0:
            m1 = m
        eqm = v == m
        c = jnp.sum(jnp.where(eqm, 1.0, 0.0), axis=1, keepdims=True)
        thr = jnp.where((cum < K_TOP) & (cum + c >= K_TOP), m, thr)
        cum = cum + c
        if j < K_TOP - 1:
            v = jnp.where(eqm, -jnp.inf, v)
    return thr, m1


def _threshold_gate(scores, tau):
    raw = scores - tau
    gate = jnp.where(raw > 0, raw, 1e-08 * jnp.exp(raw))
    eg = jnp.exp(gate) - 1.0
    thr, m1 = _top8_threshold(eg)
    eg = jnp.where(eg >= thr, eg, 0.0)
    gsum = jnp.sum(eg, axis=1, keepdims=True) + 1e-08
    gstr = jnp.tanh(m1)
    return eg / gsum * gstr


def _pool_body(x_ref, pos_ref, nt_ref, npt_ref, wt_ref, bt_ref,
               idx_ref, loss_ref, *gate_refs, N, n_tau, t_blk):
    i = pl.program_id(0)
    x = x_ref[...]
    pos = pos_ref[...]
    d0 = pos[:, 0:1] - npt_ref[0:1, :]
    d1 = pos[:, 1:2] - npt_ref[1:2, :]
    dist = d0 * d0 + d1 * d1
    scores = jnp.dot(x / KEEP, nt_ref[...],
                     preferred_element_type=jnp.float32)
    tau = jnp.dot(x, wt_ref[...], preferred_element_type=jnp.float32)
    tau = tau + bt_ref[...]

    iota_nf = jax.lax.broadcasted_iota(
        jnp.int32, (t_blk, N), 1).astype(jnp.float32)
    iota_k = jax.lax.broadcasted_iota(jnp.int32, (t_blk, K_CAND), 1)
    bigf = jnp.float32(N)

    def step(j, carry):
        d, acc_i, acc_d, acc_s = carry
        m = jnp.min(d, axis=1, keepdims=True)
        keyf = jnp.where(d == m, iota_nf, bigf)
        idxf = jnp.min(keyf, axis=1, keepdims=True)
        sel = keyf == idxf
        s = jnp.sum(jnp.where(sel, scores, 0.0), axis=1, keepdims=True)
        d = jnp.where(sel, jnp.inf, d)
        lane = iota_k == j
        acc_i = jnp.where(lane, idxf, acc_i)
        acc_d = jnp.where(lane, m, acc_d)
        acc_s = jnp.where(lane, s, acc_s)
        return d, acc_i, acc_d, acc_s

    init = (dist,
            jnp.zeros((t_blk, K_CAND), jnp.float32),
            jnp.zeros((t_blk, K_CAND), jnp.float32),
            jnp.zeros((t_blk, K_CAND), jnp.float32))
    _, cand_if, cand_d, cand_s = jax.lax.fori_loop(0, K_CAND, step, init)

    idx_ref[...] = cand_if.astype(jnp.int32)
    g0 = None
    for c in range(n_tau):
        g = _threshold_gate(cand_s, tau[:, c:c + 1])
        gate_refs[c][...] = g
        if c == 0:
            g0 = g

    part = jnp.sum(g0 * cand_d, axis=(0, 1), keepdims=True)

    @pl.when(i == 0)
    def _():
        loss_ref[...] = jnp.zeros((1, 1), jnp.float32)

    loss_ref[...] += part


def _run_pool(x2d, pos, neurons, npos, wt, bt, *, n_tau, t_blk):
    N = neurons.shape[0]
    nt = neurons.T
    npt = npos.T
    bt2 = bt.reshape(1, n_tau)
    grid = (SEQ // t_blk,)
    out_shape = ([jax.ShapeDtypeStruct((SEQ, K_CAND), jnp.int32),
                  jax.ShapeDtypeStruct((1, 1), jnp.float32)] +
                 [jax.ShapeDtypeStruct((SEQ, K_CAND), jnp.float32)] * n_tau)
    in_specs = [
        pl.BlockSpec((t_blk, D_MODEL), lambda i: (i, 0)),
        pl.BlockSpec((t_blk, POS_DIM), lambda i: (i, 0)),
        pl.BlockSpec((D_MODEL, N), lambda i: (0, 0)),
        pl.BlockSpec((POS_DIM, N), lambda i: (0, 0)),
        pl.BlockSpec((D_MODEL, n_tau), lambda i: (0, 0)),
        pl.BlockSpec((1, n_tau), lambda i: (0, 0)),
    ]
    out_specs = ([pl.BlockSpec((t_blk, K_CAND), lambda i: (i, 0)),
                  pl.BlockSpec((1, 1), lambda i: (0, 0))] +
                 [pl.BlockSpec((t_blk, K_CAND), lambda i: (i, 0))] * n_tau)
    fn = pl.pallas_call(
        functools.partial(_pool_body, N=N, n_tau=n_tau, t_blk=t_blk),
        grid=grid,
        in_specs=in_specs,
        out_specs=out_specs,
        out_shape=out_shape,
        compiler_params=pltpu.CompilerParams(
            dimension_semantics=("arbitrary",)),
    )
    return fn(x2d, pos, nt, npt, wt, bt2)


def kernel(x, qk_neurons, v_neurons, know_neurons, neuron_pos, W_pos_qk,
           b_pos_qk, W_pos_v, b_pos_v, W_pos_know, b_pos_know, W_tau_attn,
           b_tau_attn, W_tau_know, b_tau_know, deterministic):
    del deterministic
    n_qk = qk_neurons.shape[0]
    n_v = v_neurons.shape[0]
    x2d = x.reshape(SEQ, D_MODEL)

    pos_qk = (x @ W_pos_qk + b_pos_qk).reshape(SEQ, POS_DIM)
    pos_v = (x @ W_pos_v + b_pos_v).reshape(SEQ, POS_DIM)
    pos_know = (x @ W_pos_know + b_pos_know).reshape(SEQ, POS_DIM)

    idx_qk, loss_qk, gate_q, gate_k = _run_pool(
        x2d, pos_qk, qk_neurons, neuron_pos[:n_qk],
        W_tau_attn[:, 0:2], b_tau_attn[0:2], n_tau=2, t_blk=256)
    idx_v, loss_v, gate_v = _run_pool(
        x2d, pos_v, v_neurons, neuron_pos[n_qk:n_qk + n_v],
        W_tau_attn[:, 2:3], b_tau_attn[2:3], n_tau=1, t_blk=256)
    idx_know, loss_know, gate_know = _run_pool(
        x2d, pos_know, know_neurons, neuron_pos[n_qk + n_v:],
        W_tau_know, b_tau_know, n_tau=1, t_blk=256)

    denom = jnp.float32(SEQ * K_CAND)
    pos_loss_attn = (loss_qk[0, 0] + loss_v[0, 0]) / denom
    pos_loss_know = loss_know[0, 0] / denom

    to3 = lambda a: a.reshape(1, SEQ, K_CAND)
    return (to3(gate_q), to3(gate_k), to3(gate_v), to3(idx_qk), to3(idx_v),
            pos_loss_attn, to3(gate_know), to3(idx_know), pos_loss_know)


# speedup vs baseline: 6.0268x; 1.2830x over previous
import functools

import jax
import jax.numpy as jnp
from jax.experimental import pallas as pl
from jax.experimental.pallas import tpu as pltpu

D_MODEL = 768
POS_DIM = 2
K_CAND = 24
K_TOP = 8
KEEP = 0.9
SEQ = 2048


def _top8_threshold(eg):
    v = eg
    cum = jnp.zeros(eg.shape[:1] + (1,), jnp.float32)
    thr = jnp.zeros(eg.shape[:1] + (1,), jnp.float32)
    m1 = None
    for j in range(K_TOP):
        m = jnp.max(v, axis=1, keepdims=True)
        if j == 0:
            m1 = m
        eqm = v == m
        c = jnp.sum(jnp.where(eqm, 1.0, 0.0), axis=1, keepdims=True)
        thr = jnp.where((cum < K_TOP) & (cum + c >= K_TOP), m, thr)
        cum = cum + c
        if j < K_TOP - 1:
            v = jnp.where(eqm, -jnp.inf, v)
    return thr, m1


def _threshold_gate(scores, tau):
    raw = scores - tau
    gate = jnp.where(raw > 0, raw, 1e-08 * jnp.exp(raw))
    eg = jnp.exp(gate) - 1.0
    thr, m1 = _top8_threshold(eg)
    eg = jnp.where(eg >= thr, eg, 0.0)
    gsum = jnp.sum(eg, axis=1, keepdims=True) + 1e-08
    gstr = jnp.tanh(m1)
    return eg / gsum * gstr


def _pool_body(x_ref, pos_ref, nt_ref, npt_ref, wt_ref, bt_ref,
               idx_ref, loss_ref, *gate_refs, N, n_tau, t_blk):
    i = pl.program_id(0)
    x = x_ref[...]
    pos = pos_ref[...]
    d0 = pos[:, 0:1] - npt_ref[0:1, :]
    d1 = pos[:, 1:2] - npt_ref[1:2, :]
    dist = d0 * d0 + d1 * d1
    scores = jnp.dot(x / KEEP, nt_ref[...],
                     preferred_element_type=jnp.float32)
    tau = jnp.dot(x, wt_ref[...], preferred_element_type=jnp.float32)
    tau = tau + bt_ref[...]

    iota_nf = jax.lax.broadcasted_iota(
        jnp.int32, (t_blk, N), 1).astype(jnp.float32)
    bigf = jnp.float32(N)

    iota_k = jax.lax.broadcasted_iota(jnp.int32, (t_blk, K_CAND), 1)
    unroll = 4

    def step(j, carry):
        d, acc_i, acc_d, acc_s = carry
        for u in range(unroll):
            m = jnp.min(d, axis=1, keepdims=True)
            keyf = jnp.where(d == m, iota_nf, bigf)
            idxf = jnp.min(keyf, axis=1, keepdims=True)
            sel = keyf == idxf
            s = jnp.sum(jnp.where(sel, scores, 0.0), axis=1, keepdims=True)
            d = jnp.where(sel, jnp.inf, d)
            lane = iota_k == j * unroll + u
            acc_i = jnp.where(lane, idxf, acc_i)
            acc_d = jnp.where(lane, m, acc_d)
            acc_s = jnp.where(lane, s, acc_s)
        return d, acc_i, acc_d, acc_s

    init = (dist,
            jnp.zeros((t_blk, K_CAND), jnp.float32),
            jnp.zeros((t_blk, K_CAND), jnp.float32),
            jnp.zeros((t_blk, K_CAND), jnp.float32))
    _, cand_if, cand_d, cand_s = jax.lax.fori_loop(
        0, K_CAND // unroll, step, init)

    idx_ref[...] = cand_if.astype(jnp.int32)
    g0 = None
    for c in range(n_tau):
        g = _threshold_gate(cand_s, tau[:, c:c + 1])
        gate_refs[c][...] = g
        if c == 0:
            g0 = g

    part = jnp.sum(g0 * cand_d, axis=(0, 1), keepdims=True)

    @pl.when(i == 0)
    def _():
        loss_ref[...] = jnp.zeros((1, 1), jnp.float32)

    loss_ref[...] += part


def _run_pool(x2d, pos, neurons, npos, wt, bt, *, n_tau, t_blk):
    N = neurons.shape[0]
    nt = neurons.T
    npt = npos.T
    bt2 = bt.reshape(1, n_tau)
    grid = (SEQ // t_blk,)
    out_shape = ([jax.ShapeDtypeStruct((SEQ, K_CAND), jnp.int32),
                  jax.ShapeDtypeStruct((1, 1), jnp.float32)] +
                 [jax.ShapeDtypeStruct((SEQ, K_CAND), jnp.float32)] * n_tau)
    in_specs = [
        pl.BlockSpec((t_blk, D_MODEL), lambda i: (i, 0)),
        pl.BlockSpec((t_blk, POS_DIM), lambda i: (i, 0)),
        pl.BlockSpec((D_MODEL, N), lambda i: (0, 0)),
        pl.BlockSpec((POS_DIM, N), lambda i: (0, 0)),
        pl.BlockSpec((D_MODEL, n_tau), lambda i: (0, 0)),
        pl.BlockSpec((1, n_tau), lambda i: (0, 0)),
    ]
    out_specs = ([pl.BlockSpec((t_blk, K_CAND), lambda i: (i, 0)),
                  pl.BlockSpec((1, 1), lambda i: (0, 0))] +
                 [pl.BlockSpec((t_blk, K_CAND), lambda i: (i, 0))] * n_tau)
    fn = pl.pallas_call(
        functools.partial(_pool_body, N=N, n_tau=n_tau, t_blk=t_blk),
        grid=grid,
        in_specs=in_specs,
        out_specs=out_specs,
        out_shape=out_shape,
        compiler_params=pltpu.CompilerParams(
            dimension_semantics=("arbitrary",)),
    )
    return fn(x2d, pos, nt, npt, wt, bt2)


def kernel(x, qk_neurons, v_neurons, know_neurons, neuron_pos, W_pos_qk,
           b_pos_qk, W_pos_v, b_pos_v, W_pos_know, b_pos_know, W_tau_attn,
           b_tau_attn, W_tau_know, b_tau_know, deterministic):
    del deterministic
    n_qk = qk_neurons.shape[0]
    n_v = v_neurons.shape[0]
    x2d = x.reshape(SEQ, D_MODEL)

    pos_qk = (x @ W_pos_qk + b_pos_qk).reshape(SEQ, POS_DIM)
    pos_v = (x @ W_pos_v + b_pos_v).reshape(SEQ, POS_DIM)
    pos_know = (x @ W_pos_know + b_pos_know).reshape(SEQ, POS_DIM)

    idx_qk, loss_qk, gate_q, gate_k = _run_pool(
        x2d, pos_qk, qk_neurons, neuron_pos[:n_qk],
        W_tau_attn[:, 0:2], b_tau_attn[0:2], n_tau=2, t_blk=256)
    idx_v, loss_v, gate_v = _run_pool(
        x2d, pos_v, v_neurons, neuron_pos[n_qk:n_qk + n_v],
        W_tau_attn[:, 2:3], b_tau_attn[2:3], n_tau=1, t_blk=256)
    idx_know, loss_know, gate_know = _run_pool(
        x2d, pos_know, know_neurons, neuron_pos[n_qk + n_v:],
        W_tau_know, b_tau_know, n_tau=1, t_blk=128)

    denom = jnp.float32(SEQ * K_CAND)
    pos_loss_attn = (loss_qk[0, 0] + loss_v[0, 0]) / denom
    pos_loss_know = loss_know[0, 0] / denom

    to3 = lambda a: a.reshape(1, SEQ, K_CAND)
    return (to3(gate_q), to3(gate_k), to3(gate_v), to3(idx_qk), to3(idx_v),
            pos_loss_attn, to3(gate_know), to3(idx_know), pos_loss_know)


# speedup vs baseline: 6.3416x; 1.0522x over previous
import functools

import jax
import jax.numpy as jnp
from jax.experimental import pallas as pl
from jax.experimental.pallas import tpu as pltpu

D_MODEL = 768
POS_DIM = 2
K_CAND = 24
K_TOP = 8
KEEP = 0.9
SEQ = 2048


def _top8_threshold(eg):
    v = eg
    cum = jnp.zeros(eg.shape[:1] + (1,), jnp.float32)
    thr = jnp.zeros(eg.shape[:1] + (1,), jnp.float32)
    m1 = None
    for j in range(K_TOP):
        m = jnp.max(v, axis=1, keepdims=True)
        if j == 0:
            m1 = m
        eqm = v == m
        c = jnp.sum(jnp.where(eqm, 1.0, 0.0), axis=1, keepdims=True)
        thr = jnp.where((cum < K_TOP) & (cum + c >= K_TOP), m, thr)
        cum = cum + c
        if j < K_TOP - 1:
            v = jnp.where(eqm, -jnp.inf, v)
    return thr, m1


def _threshold_gate(scores, tau):
    raw = scores - tau
    gate = jnp.where(raw > 0, raw, 1e-08 * jnp.exp(raw))
    eg = jnp.exp(gate) - 1.0
    thr, m1 = _top8_threshold(eg)
    eg = jnp.where(eg >= thr, eg, 0.0)
    gsum = jnp.sum(eg, axis=1, keepdims=True) + 1e-08
    gstr = jnp.tanh(m1)
    return eg / gsum * gstr


def _pool_body(x_ref, pos_ref, nt_ref, npt_ref, wt_ref, bt_ref,
               idx_ref, loss_ref, *gate_refs, N, n_tau, t_blk):
    i = pl.program_id(0)
    x = x_ref[...]
    pos = pos_ref[...]
    d0 = pos[:, 0:1] - npt_ref[0:1, :]
    d1 = pos[:, 1:2] - npt_ref[1:2, :]
    dist = d0 * d0 + d1 * d1
    scores = jnp.dot(x / KEEP, nt_ref[...],
                     preferred_element_type=jnp.float32)
    tau = jnp.dot(x, wt_ref[...], preferred_element_type=jnp.float32)
    tau = tau + bt_ref[...]

    iota_nf = jax.lax.broadcasted_iota(
        jnp.int32, (t_blk, N), 1).astype(jnp.float32)
    bigf = jnp.float32(N)

    iota_k = jax.lax.broadcasted_iota(jnp.int32, (t_blk, K_CAND), 1)
    unroll = 8

    def step(j, carry):
        d, acc_i, acc_d, acc_s = carry
        for u in range(unroll):
            m = jnp.min(d, axis=1, keepdims=True)
            keyf = jnp.where(d == m, iota_nf, bigf)
            idxf = jnp.min(keyf, axis=1, keepdims=True)
            sel = keyf == idxf
            s = jnp.sum(jnp.where(sel, scores, 0.0), axis=1, keepdims=True)
            d = jnp.where(sel, jnp.inf, d)
            lane = iota_k == j * unroll + u
            acc_i = jnp.where(lane, idxf, acc_i)
            acc_d = jnp.where(lane, m, acc_d)
            acc_s = jnp.where(lane, s, acc_s)
        return d, acc_i, acc_d, acc_s

    init = (dist,
            jnp.zeros((t_blk, K_CAND), jnp.float32),
            jnp.zeros((t_blk, K_CAND), jnp.float32),
            jnp.zeros((t_blk, K_CAND), jnp.float32))
    _, cand_if, cand_d, cand_s = jax.lax.fori_loop(
        0, K_CAND // unroll, step, init)

    idx_ref[...] = cand_if.astype(jnp.int32)
    g0 = None
    for c in range(n_tau):
        g = _threshold_gate(cand_s, tau[:, c:c + 1])
        gate_refs[c][...] = g
        if c == 0:
            g0 = g

    part = jnp.sum(g0 * cand_d, axis=(0, 1), keepdims=True)

    @pl.when(i == 0)
    def _():
        loss_ref[...] = jnp.zeros((1, 1), jnp.float32)

    loss_ref[...] += part


def _run_pool(x2d, pos, neurons, npos, wt, bt, *, n_tau, t_blk):
    N = neurons.shape[0]
    nt = neurons.T
    npt = npos.T
    bt2 = bt.reshape(1, n_tau)
    grid = (SEQ // t_blk,)
    out_shape = ([jax.ShapeDtypeStruct((SEQ, K_CAND), jnp.int32),
                  jax.ShapeDtypeStruct((1, 1), jnp.float32)] +
                 [jax.ShapeDtypeStruct((SEQ, K_CAND), jnp.float32)] * n_tau)
    in_specs = [
        pl.BlockSpec((t_blk, D_MODEL), lambda i: (i, 0)),
        pl.BlockSpec((t_blk, POS_DIM), lambda i: (i, 0)),
        pl.BlockSpec((D_MODEL, N), lambda i: (0, 0)),
        pl.BlockSpec((POS_DIM, N), lambda i: (0, 0)),
        pl.BlockSpec((D_MODEL, n_tau), lambda i: (0, 0)),
        pl.BlockSpec((1, n_tau), lambda i: (0, 0)),
    ]
    out_specs = ([pl.BlockSpec((t_blk, K_CAND), lambda i: (i, 0)),
                  pl.BlockSpec((1, 1), lambda i: (0, 0))] +
                 [pl.BlockSpec((t_blk, K_CAND), lambda i: (i, 0))] * n_tau)
    fn = pl.pallas_call(
        functools.partial(_pool_body, N=N, n_tau=n_tau, t_blk=t_blk),
        grid=grid,
        in_specs=in_specs,
        out_specs=out_specs,
        out_shape=out_shape,
        compiler_params=pltpu.CompilerParams(
            dimension_semantics=("arbitrary",)),
    )
    return fn(x2d, pos, nt, npt, wt, bt2)


def kernel(x, qk_neurons, v_neurons, know_neurons, neuron_pos, W_pos_qk,
           b_pos_qk, W_pos_v, b_pos_v, W_pos_know, b_pos_know, W_tau_attn,
           b_tau_attn, W_tau_know, b_tau_know, deterministic):
    del deterministic
    n_qk = qk_neurons.shape[0]
    n_v = v_neurons.shape[0]
    x2d = x.reshape(SEQ, D_MODEL)

    pos_qk = (x @ W_pos_qk + b_pos_qk).reshape(SEQ, POS_DIM)
    pos_v = (x @ W_pos_v + b_pos_v).reshape(SEQ, POS_DIM)
    pos_know = (x @ W_pos_know + b_pos_know).reshape(SEQ, POS_DIM)

    idx_qk, loss_qk, gate_q, gate_k = _run_pool(
        x2d, pos_qk, qk_neurons, neuron_pos[:n_qk],
        W_tau_attn[:, 0:2], b_tau_attn[0:2], n_tau=2, t_blk=256)
    idx_v, loss_v, gate_v = _run_pool(
        x2d, pos_v, v_neurons, neuron_pos[n_qk:n_qk + n_v],
        W_tau_attn[:, 2:3], b_tau_attn[2:3], n_tau=1, t_blk=256)
    idx_know, loss_know, gate_know = _run_pool(
        x2d, pos_know, know_neurons, neuron_pos[n_qk + n_v:],
        W_tau_know, b_tau_know, n_tau=1, t_blk=128)

    denom = jnp.float32(SEQ * K_CAND)
    pos_loss_attn = (loss_qk[0, 0] + loss_v[0, 0]) / denom
    pos_loss_know = loss_know[0, 0] / denom

    to3 = lambda a: a.reshape(1, SEQ, K_CAND)
    return (to3(gate_q), to3(gate_k), to3(gate_v), to3(idx_qk), to3(idx_v),
            pos_loss_attn, to3(gate_know), to3(idx_know), pos_loss_know)


# speedup vs baseline: 6.4694x; 1.0202x over previous
import functools

import jax
import jax.numpy as jnp
from jax.experimental import pallas as pl
from jax.experimental.pallas import tpu as pltpu

D_MODEL = 768
POS_DIM = 2
K_CAND = 24
K_TOP = 8
KEEP = 0.9
SEQ = 2048


def _top8_threshold(eg):
    v = eg
    cum = jnp.zeros(eg.shape[:1] + (1,), jnp.float32)
    thr = jnp.zeros(eg.shape[:1] + (1,), jnp.float32)
    m1 = None
    for j in range(K_TOP):
        m = jnp.max(v, axis=1, keepdims=True)
        if j == 0:
            m1 = m
        eqm = v == m
        c = jnp.sum(jnp.where(eqm, 1.0, 0.0), axis=1, keepdims=True)
        thr = jnp.where((cum < K_TOP) & (cum + c >= K_TOP), m, thr)
        cum = cum + c
        if j < K_TOP - 1:
            v = jnp.where(eqm, -jnp.inf, v)
    return thr, m1


def _threshold_gate(scores, tau):
    raw = scores - tau
    gate = jnp.where(raw > 0, raw, 1e-08 * jnp.exp(raw))
    eg = jnp.exp(gate) - 1.0
    thr, m1 = _top8_threshold(eg)
    eg = jnp.where(eg >= thr, eg, 0.0)
    gsum = jnp.sum(eg, axis=1, keepdims=True) + 1e-08
    gstr = jnp.tanh(m1)
    return eg / gsum * gstr


def _pool_body(x_ref, pos_ref, nt_ref, npt_ref, wt_ref, bt_ref,
               idx_ref, loss_ref, *gate_refs, N, n_tau, t_blk):
    i = pl.program_id(0)
    x = x_ref[...]
    pos = pos_ref[...]
    d0 = pos[:, 0:1] - npt_ref[0:1, :]
    d1 = pos[:, 1:2] - npt_ref[1:2, :]
    dist = d0 * d0 + d1 * d1
    scores = jnp.dot(x / KEEP, nt_ref[...],
                     preferred_element_type=jnp.float32)
    tau = jnp.dot(x, wt_ref[...], preferred_element_type=jnp.float32)
    tau = tau + bt_ref[...]

    iota_nf = jax.lax.broadcasted_iota(
        jnp.int32, (t_blk, N), 1).astype(jnp.float32)
    bigf = jnp.float32(N)

    iota_k = jax.lax.broadcasted_iota(jnp.int32, (t_blk, K_CAND), 1)
    unroll = 12

    def step(j, carry):
        d, acc_i, acc_d, acc_s = carry
        for u in range(unroll):
            m = jnp.min(d, axis=1, keepdims=True)
            keyf = jnp.where(d == m, iota_nf, bigf)
            idxf = jnp.min(keyf, axis=1, keepdims=True)
            sel = keyf == idxf
            s = jnp.sum(jnp.where(sel, scores, 0.0), axis=1, keepdims=True)
            d = jnp.where(sel, jnp.inf, d)
            lane = iota_k == j * unroll + u
            acc_i = jnp.where(lane, idxf, acc_i)
            acc_d = jnp.where(lane, m, acc_d)
            acc_s = jnp.where(lane, s, acc_s)
        return d, acc_i, acc_d, acc_s

    init = (dist,
            jnp.zeros((t_blk, K_CAND), jnp.float32),
            jnp.zeros((t_blk, K_CAND), jnp.float32),
            jnp.zeros((t_blk, K_CAND), jnp.float32))
    _, cand_if, cand_d, cand_s = jax.lax.fori_loop(
        0, K_CAND // unroll, step, init)

    idx_ref[...] = cand_if.astype(jnp.int32)
    g0 = None
    for c in range(n_tau):
        g = _threshold_gate(cand_s, tau[:, c:c + 1])
        gate_refs[c][...] = g
        if c == 0:
            g0 = g

    part = jnp.sum(g0 * cand_d, axis=(0, 1), keepdims=True)

    @pl.when(i == 0)
    def _():
        loss_ref[...] = jnp.zeros((1, 1), jnp.float32)

    loss_ref[...] += part


def _run_pool(x2d, pos, neurons, npos, wt, bt, *, n_tau, t_blk):
    N = neurons.shape[0]
    nt = neurons.T
    npt = npos.T
    bt2 = bt.reshape(1, n_tau)
    grid = (SEQ // t_blk,)
    out_shape = ([jax.ShapeDtypeStruct((SEQ, K_CAND), jnp.int32),
                  jax.ShapeDtypeStruct((1, 1), jnp.float32)] +
                 [jax.ShapeDtypeStruct((SEQ, K_CAND), jnp.float32)] * n_tau)
    in_specs = [
        pl.BlockSpec((t_blk, D_MODEL), lambda i: (i, 0)),
        pl.BlockSpec((t_blk, POS_DIM), lambda i: (i, 0)),
        pl.BlockSpec((D_MODEL, N), lambda i: (0, 0)),
        pl.BlockSpec((POS_DIM, N), lambda i: (0, 0)),
        pl.BlockSpec((D_MODEL, n_tau), lambda i: (0, 0)),
        pl.BlockSpec((1, n_tau), lambda i: (0, 0)),
    ]
    out_specs = ([pl.BlockSpec((t_blk, K_CAND), lambda i: (i, 0)),
                  pl.BlockSpec((1, 1), lambda i: (0, 0))] +
                 [pl.BlockSpec((t_blk, K_CAND), lambda i: (i, 0))] * n_tau)
    fn = pl.pallas_call(
        functools.partial(_pool_body, N=N, n_tau=n_tau, t_blk=t_blk),
        grid=grid,
        in_specs=in_specs,
        out_specs=out_specs,
        out_shape=out_shape,
        compiler_params=pltpu.CompilerParams(
            dimension_semantics=("arbitrary",)),
    )
    return fn(x2d, pos, nt, npt, wt, bt2)


def kernel(x, qk_neurons, v_neurons, know_neurons, neuron_pos, W_pos_qk,
           b_pos_qk, W_pos_v, b_pos_v, W_pos_know, b_pos_know, W_tau_attn,
           b_tau_attn, W_tau_know, b_tau_know, deterministic):
    del deterministic
    n_qk = qk_neurons.shape[0]
    n_v = v_neurons.shape[0]
    x2d = x.reshape(SEQ, D_MODEL)

    pos_qk = (x @ W_pos_qk + b_pos_qk).reshape(SEQ, POS_DIM)
    pos_v = (x @ W_pos_v + b_pos_v).reshape(SEQ, POS_DIM)
    pos_know = (x @ W_pos_know + b_pos_know).reshape(SEQ, POS_DIM)

    idx_qk, loss_qk, gate_q, gate_k = _run_pool(
        x2d, pos_qk, qk_neurons, neuron_pos[:n_qk],
        W_tau_attn[:, 0:2], b_tau_attn[0:2], n_tau=2, t_blk=256)
    idx_v, loss_v, gate_v = _run_pool(
        x2d, pos_v, v_neurons, neuron_pos[n_qk:n_qk + n_v],
        W_tau_attn[:, 2:3], b_tau_attn[2:3], n_tau=1, t_blk=256)
    idx_know, loss_know, gate_know = _run_pool(
        x2d, pos_know, know_neurons, neuron_pos[n_qk + n_v:],
        W_tau_know, b_tau_know, n_tau=1, t_blk=128)

    denom = jnp.float32(SEQ * K_CAND)
    pos_loss_attn = (loss_qk[0, 0] + loss_v[0, 0]) / denom
    pos_loss_know = loss_know[0, 0] / denom

    to3 = lambda a: a.reshape(1, SEQ, K_CAND)
    return (to3(gate_q), to3(gate_k), to3(gate_v), to3(idx_qk), to3(idx_v),
            pos_loss_attn, to3(gate_know), to3(idx_know), pos_loss_know)
